# baseline (speedup 1.0000x reference)
import functools

import jax
import jax.numpy as jnp
from jax import lax
from jax.experimental import pallas as pl
from jax.experimental.pallas import tpu as pltpu
from jax.experimental.pallas import tpu_sc as plsc

B = 4096
S = 200
VOCAB = 1000000
D = 64
DP = 128
HID = 256
NCLS = 50

NC = 2
NS = 16
L = 16
NW = NC * NS
BPW = B // NW

_mesh = plsc.VectorSubcoreMesh(core_axis_name="c", subcore_axis_name="s")

NSLOT = 2


@functools.partial(
    pl.kernel,
    out_type=jax.ShapeDtypeStruct((B, D), jnp.float32),
    mesh=_mesh,
    scratch_types=[
        pltpu.VMEM((BPW * S,), jnp.int32),
        pltpu.VMEM((NSLOT, S, DP), jnp.float32),
        pltpu.VMEM((BPW, D), jnp.float32),
        pltpu.SemaphoreType.DMA,
        pltpu.SemaphoreType.DMA,
    ],
)
def _sc_pool(x_hbm, emb_hbm, out_hbm, idx_v, buf_v, pooled_v, s0, s1):
    sems = (s0, s1)
    wid = lax.axis_index("s") * NC + lax.axis_index("c")
    base = wid * BPW
    pltpu.sync_copy(x_hbm.at[pl.ds(base * S, BPW * S)], idx_v)

    def start_elem(i, b):
        pltpu.async_copy(
            emb_hbm.at[idx_v.at[pl.ds(i * S, S)]], buf_v.at[b], sems[b]
        )

    def wait_elem(i, b):
        pltpu.make_async_copy(
            emb_hbm.at[idx_v.at[pl.ds(i * S, S)]], buf_v.at[b], sems[b]
        ).wait()

    RU = 8

    def acc_elem(i, b):
        def body(t, acc):
            r0 = t * RU
            nxt = list(acc)
            for u in range(RU):
                for c in range(4):
                    k = (u % 2) * 4 + c
                    nxt[k] = nxt[k] + buf_v[b, r0 + u, pl.ds(c * L, L)]
            return tuple(nxt)

        zero = jnp.zeros((L,), jnp.float32)
        acc = lax.fori_loop(0, S // RU, body, (zero,) * 8)
        for c in range(4):
            pooled_v[i, pl.ds(c * L, L)] = acc[c] + acc[4 + c]

    for b in range(NSLOT):
        start_elem(b, b)

    @pl.loop(0, BPW, step=NSLOT)
    def _(i):
        for b in range(NSLOT):
            wait_elem(i + b, b)
            acc_elem(i + b, b)

            @pl.when(i + b + NSLOT < BPW)
            def _(b=b):
                start_elem(i + b + NSLOT, b)

    pltpu.sync_copy(pooled_v, out_hbm.at[pl.ds(base, BPW)])


CB = 4000


def _pad_body(e_ref, o_ref):
    o_ref[:, :D] = e_ref[...]


def _tc_pad(emb):
    return pl.pallas_call(
        _pad_body,
        grid=(VOCAB // CB,),
        in_specs=[pl.BlockSpec((CB, D), lambda i: (i, 0))],
        out_specs=pl.BlockSpec((CB, DP), lambda i: (i, 0)),
        out_shape=jax.ShapeDtypeStruct((VOCAB, DP), jnp.float32),
    )(emb)


def _mlp_body(p_ref, w1_ref, b1_ref, w2_ref, b2_ref, o_ref):
    p = p_ref[...] * (1.0 / S)
    h = jnp.dot(p, w1_ref[...], preferred_element_type=jnp.float32)
    h = jnp.maximum(h + b1_ref[...], 0.0)
    o_ref[...] = jnp.dot(h, w2_ref[...], preferred_element_type=jnp.float32) + b2_ref[...]


def _mlp(pooled, W1, b1, W2, b2):
    BT = 512
    return pl.pallas_call(
        _mlp_body,
        grid=(B // BT,),
        in_specs=[
            pl.BlockSpec((BT, D), lambda i: (i, 0)),
            pl.BlockSpec((D, HID), lambda i: (0, 0)),
            pl.BlockSpec((1, HID), lambda i: (0, 0)),
            pl.BlockSpec((HID, NCLS), lambda i: (0, 0)),
            pl.BlockSpec((1, NCLS), lambda i: (0, 0)),
        ],
        out_specs=pl.BlockSpec((BT, NCLS), lambda i: (i, 0)),
        out_shape=jax.ShapeDtypeStruct((B, NCLS), jnp.float32),
    )(pooled, W1, b1.reshape(1, HID), W2, b2.reshape(1, NCLS))


def kernel(x, emb, W1, b1, W2, b2):
    x1 = x.astype(jnp.int32).reshape(B * S)
    emb_p = _tc_pad(emb)
    pooled = _sc_pool(x1, emb_p)
    return _mlp(pooled, W1, b1, W2, b2)


# speedup vs baseline: 1.8425x; 1.8425x over previous
import functools

import jax
import jax.numpy as jnp
from jax import lax
from jax.experimental import pallas as pl
from jax.experimental.pallas import tpu as pltpu
from jax.experimental.pallas import tpu_sc as plsc

B = 4096
S = 200
VOCAB = 1000000
D = 64
DP = 128
HID = 256
NCLS = 50

NC = 2
NS = 16
L = 16
NW = NC * NS
BPW = B // NW

_mesh = plsc.VectorSubcoreMesh(core_axis_name="c", subcore_axis_name="s")

NSLOT = 2


@functools.partial(
    pl.kernel,
    out_type=jax.ShapeDtypeStruct((B, D), jnp.float32),
    mesh=_mesh,
    scratch_types=[
        pltpu.VMEM((BPW * S,), jnp.int32),
        pltpu.VMEM((NSLOT, S, DP), jnp.float32),
        pltpu.VMEM((BPW, D), jnp.float32),
        pltpu.SemaphoreType.DMA,
        pltpu.SemaphoreType.DMA,
    ],
)
def _sc_pool(x_hbm, emb_hbm, out_hbm, idx_v, buf_v, pooled_v, s0, s1):
    sems = (s0, s1)
    wid = lax.axis_index("s") * NC + lax.axis_index("c")
    base = wid * BPW
    pltpu.sync_copy(x_hbm.at[pl.ds(base * S, BPW * S)], idx_v)

    def start_elem(i, b):
        pltpu.async_copy(
            emb_hbm.at[idx_v.at[pl.ds(i * S, S)]], buf_v.at[b], sems[b]
        )

    def wait_elem(i, b):
        pltpu.make_async_copy(
            emb_hbm.at[idx_v.at[pl.ds(i * S, S)]], buf_v.at[b], sems[b]
        ).wait()

    RU = 8

    def acc_elem(i, b):
        def body(t, acc):
            r0 = t * RU
            nxt = list(acc)
            for u in range(RU):
                for c in range(4):
                    k = (u % 2) * 4 + c
                    nxt[k] = nxt[k] + buf_v[b, r0 + u, pl.ds(c * L, L)]
            return tuple(nxt)

        zero = jnp.zeros((L,), jnp.float32)
        acc = lax.fori_loop(0, S // RU, body, (zero,) * 8)
        for c in range(4):
            pooled_v[i, pl.ds(c * L, L)] = acc[c] + acc[4 + c]

    for b in range(NSLOT):
        start_elem(b, b)

    @pl.loop(0, BPW, step=NSLOT)
    def _(i):
        for b in range(NSLOT):
            wait_elem(i + b, b)
            acc_elem(i + b, b)

            @pl.when(i + b + NSLOT < BPW)
            def _(b=b):
                start_elem(i + b + NSLOT, b)

    pltpu.sync_copy(pooled_v, out_hbm.at[pl.ds(base, BPW)])


CB = 8192


def _relayout_body(et_ref, o_ref):
    o_ref[:, :D] = et_ref[...].T
    o_ref[:, D:] = jnp.zeros((CB, DP - D), jnp.float32)


def _tc_relayout(emb_t):
    return pl.pallas_call(
        _relayout_body,
        grid=(pl.cdiv(VOCAB, CB),),
        in_specs=[pl.BlockSpec((D, CB), lambda i: (0, i))],
        out_specs=pl.BlockSpec((CB, DP), lambda i: (i, 0)),
        out_shape=jax.ShapeDtypeStruct((VOCAB, DP), jnp.float32),
    )(emb_t)


def _mlp_body(p_ref, w1_ref, b1_ref, w2_ref, b2_ref, o_ref):
    p = p_ref[...] * (1.0 / S)
    h = jnp.dot(p, w1_ref[...], preferred_element_type=jnp.float32)
    h = jnp.maximum(h + b1_ref[...], 0.0)
    o_ref[...] = jnp.dot(h, w2_ref[...], preferred_element_type=jnp.float32) + b2_ref[...]


def _mlp(pooled, W1, b1, W2, b2):
    BT = 512
    return pl.pallas_call(
        _mlp_body,
        grid=(B // BT,),
        in_specs=[
            pl.BlockSpec((BT, D), lambda i: (i, 0)),
            pl.BlockSpec((D, HID), lambda i: (0, 0)),
            pl.BlockSpec((1, HID), lambda i: (0, 0)),
            pl.BlockSpec((HID, NCLS), lambda i: (0, 0)),
            pl.BlockSpec((1, NCLS), lambda i: (0, 0)),
        ],
        out_specs=pl.BlockSpec((BT, NCLS), lambda i: (i, 0)),
        out_shape=jax.ShapeDtypeStruct((B, NCLS), jnp.float32),
    )(pooled, W1, b1.reshape(1, HID), W2, b2.reshape(1, NCLS))


def kernel(x, emb, W1, b1, W2, b2):
    x1 = x.astype(jnp.int32).reshape(B * S)
    emb_p = _tc_relayout(emb.T)
    pooled = _sc_pool(x1, emb_p)
    return _mlp(pooled, W1, b1, W2, b2)


# speedup vs baseline: 1.9227x; 1.0435x over previous
import functools

import jax
import jax.numpy as jnp
from jax import lax
from jax.experimental import pallas as pl
from jax.experimental.pallas import tpu as pltpu
from jax.experimental.pallas import tpu_sc as plsc

B = 4096
S = 200
VOCAB = 1000000
D = 64
DP = 128
HID = 256
NCLS = 50

NC = 2
NS = 16
L = 16
NW = NC * NS
BPW = B // NW

_mesh = plsc.VectorSubcoreMesh(core_axis_name="c", subcore_axis_name="s")

NSLOT = 2


@functools.partial(
    pl.kernel,
    out_type=jax.ShapeDtypeStruct((B, D), jnp.float32),
    mesh=_mesh,
    scratch_types=[
        pltpu.VMEM((BPW * S,), jnp.int32),
        pltpu.VMEM((NSLOT, S, DP), jnp.float32),
        pltpu.VMEM((BPW, D), jnp.float32),
        pltpu.SemaphoreType.DMA,
        pltpu.SemaphoreType.DMA,
    ],
)
def _sc_pool(x_hbm, emb_hbm, out_hbm, idx_v, buf_v, pooled_v, s0, s1):
    sems = (s0, s1)
    wid = lax.axis_index("s") * NC + lax.axis_index("c")
    base = wid * BPW
    pltpu.sync_copy(x_hbm.at[pl.ds(base * S, BPW * S)], idx_v)

    def start_elem(i, b):
        pltpu.async_copy(
            emb_hbm.at[idx_v.at[pl.ds(i * S, S)]], buf_v.at[b], sems[b]
        )

    def wait_elem(i, b):
        pltpu.make_async_copy(
            emb_hbm.at[idx_v.at[pl.ds(i * S, S)]], buf_v.at[b], sems[b]
        ).wait()

    RU = 8

    def acc_elem(i, b):
        def body(t, acc):
            r0 = t * RU
            nxt = list(acc)
            for u in range(RU):
                for c in range(4):
                    k = (u % 2) * 4 + c
                    nxt[k] = nxt[k] + buf_v[b, r0 + u, pl.ds(c * L, L)]
            return tuple(nxt)

        zero = jnp.zeros((L,), jnp.float32)
        acc = lax.fori_loop(0, S // RU, body, (zero,) * 8)
        for c in range(4):
            pooled_v[i, pl.ds(c * L, L)] = acc[c] + acc[4 + c]

    for b in range(NSLOT):
        start_elem(b, b)

    @pl.loop(0, BPW, step=NSLOT)
    def _(i):
        for b in range(NSLOT):
            wait_elem(i + b, b)
            acc_elem(i + b, b)

            @pl.when(i + b + NSLOT < BPW)
            def _(b=b):
                start_elem(i + b + NSLOT, b)

    pltpu.sync_copy(pooled_v, out_hbm.at[pl.ds(base, BPW)])


CB = 16384


def _relayout_body(et_ref, o_ref):
    o_ref[:, :D] = et_ref[...].T
    o_ref[:, D:] = jnp.zeros((CB, DP - D), jnp.float32)


def _tc_relayout(emb_t):
    return pl.pallas_call(
        _relayout_body,
        grid=(pl.cdiv(VOCAB, CB),),
        in_specs=[pl.BlockSpec((D, CB), lambda i: (0, i))],
        out_specs=pl.BlockSpec((CB, DP), lambda i: (i, 0)),
        out_shape=jax.ShapeDtypeStruct((VOCAB, DP), jnp.float32),
    )(emb_t)


def _mlp_body(p_ref, w1_ref, b1_ref, w2_ref, b2_ref, o_ref):
    p = p_ref[...] * (1.0 / S)
    h = jnp.dot(p, w1_ref[...], preferred_element_type=jnp.float32)
    h = jnp.maximum(h + b1_ref[...], 0.0)
    o_ref[...] = jnp.dot(h, w2_ref[...], preferred_element_type=jnp.float32) + b2_ref[...]


def _mlp(pooled, W1, b1, W2, b2):
    BT = 512
    return pl.pallas_call(
        _mlp_body,
        grid=(B // BT,),
        in_specs=[
            pl.BlockSpec((BT, D), lambda i: (i, 0)),
            pl.BlockSpec((D, HID), lambda i: (0, 0)),
            pl.BlockSpec((1, HID), lambda i: (0, 0)),
            pl.BlockSpec((HID, NCLS), lambda i: (0, 0)),
            pl.BlockSpec((1, NCLS), lambda i: (0, 0)),
        ],
        out_specs=pl.BlockSpec((BT, NCLS), lambda i: (i, 0)),
        out_shape=jax.ShapeDtypeStruct((B, NCLS), jnp.float32),
    )(pooled, W1, b1.reshape(1, HID), W2, b2.reshape(1, NCLS))


def kernel(x, emb, W1, b1, W2, b2):
    x1 = x.astype(jnp.int32).reshape(B * S)
    emb_p = _tc_relayout(emb.T)
    pooled = _sc_pool(x1, emb_p)
    return _mlp(pooled, W1, b1, W2, b2)


# speedup vs baseline: 1.9408x; 1.0094x over previous
import functools

import jax
import jax.numpy as jnp
from jax import lax
from jax.experimental import pallas as pl
from jax.experimental.pallas import tpu as pltpu
from jax.experimental.pallas import tpu_sc as plsc

B = 4096
S = 200
VOCAB = 1000000
D = 64
DP = 128
HID = 256
NCLS = 50

NC = 2
NS = 16
L = 16
NW = NC * NS
BPW = B // NW

_mesh = plsc.VectorSubcoreMesh(core_axis_name="c", subcore_axis_name="s")

NSLOT = 2


@functools.partial(
    pl.kernel,
    out_type=jax.ShapeDtypeStruct((B, D), jnp.float32),
    mesh=_mesh,
    scratch_types=[
        pltpu.VMEM((BPW * S,), jnp.int32),
        pltpu.VMEM((NSLOT, S, DP), jnp.float32),
        pltpu.VMEM((BPW, D), jnp.float32),
        pltpu.SemaphoreType.DMA,
        pltpu.SemaphoreType.DMA,
    ],
)
def _sc_pool(x_hbm, emb_hbm, out_hbm, idx_v, buf_v, pooled_v, s0, s1):
    sems = (s0, s1)
    wid = lax.axis_index("s") * NC + lax.axis_index("c")
    base = wid * BPW
    pltpu.sync_copy(x_hbm.at[pl.ds(base * S, BPW * S)], idx_v)

    def start_elem(i, b):
        pltpu.async_copy(
            emb_hbm.at[idx_v.at[pl.ds(i * S, S)]], buf_v.at[b], sems[b]
        )

    def wait_elem(i, b):
        pltpu.make_async_copy(
            emb_hbm.at[idx_v.at[pl.ds(i * S, S)]], buf_v.at[b], sems[b]
        ).wait()

    RU = 8

    def acc_elem(i, b):
        def body(t, acc):
            r0 = t * RU
            nxt = list(acc)
            for u in range(RU):
                for c in range(4):
                    k = (u % 2) * 4 + c
                    nxt[k] = nxt[k] + buf_v[b, r0 + u, pl.ds(c * L, L)]
            return tuple(nxt)

        zero = jnp.zeros((L,), jnp.float32)
        acc = lax.fori_loop(0, S // RU, body, (zero,) * 8)
        for c in range(4):
            pooled_v[i, pl.ds(c * L, L)] = acc[c] + acc[4 + c]

    for b in range(NSLOT):
        start_elem(b, b)

    @pl.loop(0, BPW, step=NSLOT)
    def _(i):
        for b in range(NSLOT):
            wait_elem(i + b, b)
            acc_elem(i + b, b)

            @pl.when(i + b + NSLOT < BPW)
            def _(b=b):
                start_elem(i + b + NSLOT, b)

    pltpu.sync_copy(pooled_v, out_hbm.at[pl.ds(base, BPW)])


CB = 32768


def _relayout_body(et_ref, o_ref):
    o_ref[:, :D] = et_ref[...].T
    o_ref[:, D:] = jnp.zeros((CB, DP - D), jnp.float32)


def _tc_relayout(emb_t):
    return pl.pallas_call(
        _relayout_body,
        grid=(pl.cdiv(VOCAB, CB),),
        in_specs=[pl.BlockSpec((D, CB), lambda i: (0, i))],
        out_specs=pl.BlockSpec((CB, DP), lambda i: (i, 0)),
        out_shape=jax.ShapeDtypeStruct((VOCAB, DP), jnp.float32),
    )(emb_t)


def _mlp_body(p_ref, w1_ref, b1_ref, w2_ref, b2_ref, o_ref):
    p = p_ref[...] * (1.0 / S)
    h = jnp.dot(p, w1_ref[...], preferred_element_type=jnp.float32)
    h = jnp.maximum(h + b1_ref[...], 0.0)
    o_ref[...] = jnp.dot(h, w2_ref[...], preferred_element_type=jnp.float32) + b2_ref[...]


def _mlp(pooled, W1, b1, W2, b2):
    BT = 512
    return pl.pallas_call(
        _mlp_body,
        grid=(B // BT,),
        in_specs=[
            pl.BlockSpec((BT, D), lambda i: (i, 0)),
            pl.BlockSpec((D, HID), lambda i: (0, 0)),
            pl.BlockSpec((1, HID), lambda i: (0, 0)),
            pl.BlockSpec((HID, NCLS), lambda i: (0, 0)),
            pl.BlockSpec((1, NCLS), lambda i: (0, 0)),
        ],
        out_specs=pl.BlockSpec((BT, NCLS), lambda i: (i, 0)),
        out_shape=jax.ShapeDtypeStruct((B, NCLS), jnp.float32),
    )(pooled, W1, b1.reshape(1, HID), W2, b2.reshape(1, NCLS))


def kernel(x, emb, W1, b1, W2, b2):
    x1 = x.astype(jnp.int32).reshape(B * S)
    emb_p = _tc_relayout(emb.T)
    pooled = _sc_pool(x1, emb_p)
    return _mlp(pooled, W1, b1, W2, b2)
